# Initial kernel scaffold; baseline (speedup 1.0000x reference)
import jax
import jax.numpy as jnp
from jax.experimental import pallas as pl


def kernel(x, params):
    raise NotImplementedError("write your pallas kernel here")


# speedup vs baseline: 525.6782x; 525.6782x over previous
import jax
import jax.numpy as jnp
from jax.experimental import pallas as pl

_N = 512
_F0 = 64
_F1 = 16
_NB = 8
_RMAX = 5.0
_AVG = 511.0
_PI = 3.141592653589793


def _dot(a, b):
    return jax.lax.dot_general(
        a, b, (((1,), (0,)), ((), ())),
        preferred_element_type=jnp.float32,
        precision=jax.lax.Precision.HIGHEST)


def _mace_body(x_ref, xt_ref, *refs):
    (wemb,
     wr0_0, wr1_0, wr2_0, wsc_0, ws0_0, wu0_0, wn_0, wg_0, wh1_0, wu1_0,
     wr0_1, wr1_1, wr2_1, wsc_1, ws0_1, wu0_1, wn_1, wg_1, wh1_1, wu1_1,
     wro0, wro1, out0_ref, out1_ref) = refs
    f32 = jnp.float32

    vec = []
    for c in range(3):
        col = x_ref[:, c:c + 1]
        row = xt_ref[c:c + 1, :]
        vec.append(col - row)
    d2 = vec[0] * vec[0] + vec[1] * vec[1] + vec[2] * vec[2] + 1e-12
    d = jnp.sqrt(d2)
    dinv = 1.0 / (d + 1e-9)
    u = jnp.clip(d / _RMAX, 0.0, 1.0 - 1e-6)
    env = jnp.where(d < _RMAX, jnp.exp(1.0 - 1.0 / (1.0 - u * u)), 0.0)
    ii = jax.lax.broadcasted_iota(jnp.int32, (_N, _N), 0)
    jj = jax.lax.broadcasted_iota(jnp.int32, (_N, _N), 1)
    pref = jnp.where(ii == jj, 0.0, jnp.sqrt(2.0 / _RMAX) * env * dinv)
    rb = [pref * jnp.sin(((b + 1) * _PI / _RMAX) * d) for b in range(_NB)]
    yhat = [v * dinv for v in vec]

    h0e = wemb[:]
    sc0 = _dot(h0e, wsc_0[:])
    wr0v, wr1v = wr0_0[:], wr1_0[:]
    S0 = jnp.zeros((_N, _F0), f32)
    t1 = [jnp.zeros((_N, _F1), f32) for _ in range(3)]
    for b in range(_NB):
        rbs = jnp.sum(rb[b], axis=1, keepdims=True)
        S0 = S0 + rbs * wr0v[b:b + 1, :]
        for c in range(3):
            rys = jnp.sum(rb[b] * yhat[c], axis=1, keepdims=True)
            t1[c] = t1[c] + rys * wr1v[b:b + 1, :]
    H0 = S0 * h0e * (1.0 / _AVG)
    H1 = [t1[c] * sc0 * (1.0 / _AVG) for c in range(3)]
    norms = H1[0] * H1[0] + H1[1] * H1[1] + H1[2] * H1[2]
    pre = (_dot(h0e, ws0_0[:]) + _dot(H0, wu0_0[:]) + _dot(norms, wn_0[:]))
    h0 = pre * jax.nn.sigmoid(pre)
    gate = jax.nn.sigmoid(_dot(h0, wg_0[:]))
    h1 = [_dot(H1[c], wu1_0[:]) * gate for c in range(3)]

    sc = _dot(h0, wsc_1[:])
    rhs = jnp.concatenate([h0, h1[0], h1[1], h1[2]], axis=1)
    wr0v, wr1v, wr2v = wr0_1[:], wr1_1[:], wr2_1[:]
    H0 = jnp.zeros((_N, _F0), f32)
    t1 = [jnp.zeros((_N, _F1), f32) for _ in range(3)]
    t2 = [jnp.zeros((_N, _F1), f32) for _ in range(3)]
    for b in range(_NB):
        Tb = _dot(rb[b], rhs)
        H0 = H0 + wr0v[b:b + 1, :] * Tb[:, :_F0]
        for c in range(3):
            lo = _F0 + _F1 * c
            t2[c] = t2[c] + wr2v[b:b + 1, :] * Tb[:, lo:lo + _F1]
            Mbc = _dot(rb[b] * yhat[c], sc)
            t1[c] = t1[c] + wr1v[b:b + 1, :] * Mbc
    H0 = H0 * (1.0 / _AVG)
    H1 = [(t1[c] + t2[c]) * (1.0 / _AVG) for c in range(3)]
    norms = H1[0] * H1[0] + H1[1] * H1[1] + H1[2] * H1[2]
    pre = (_dot(h0, ws0_1[:]) + _dot(H0, wu0_1[:]) + _dot(norms, wn_1[:]))
    h0 = pre * jax.nn.sigmoid(pre)
    gate = jax.nn.sigmoid(_dot(h0, wg_1[:]))
    h1 = [(_dot(h1[c], wh1_1[:]) + _dot(H1[c], wu1_1[:])) * gate
          for c in range(3)]

    out0_ref[:, :] = _dot(h0, wro0[:])
    for c in range(3):
        com_c = jnp.mean(x_ref[:, c])
        out1_ref[c] = _dot(h1[c], wro1[:]) + com_c


def kernel(x, params):
    xt = x.T
    names = ['Wr0', 'Wr1', 'Wr2', 'Wsc', 'Ws0', 'Wu0', 'Wn', 'Wg', 'Wh1',
             'Wu1']
    args = [x, xt, params['Wemb']]
    for i in range(2):
        args += [params[nm + '_' + str(i)] for nm in names]
    args += [params['Wro0'], params['Wro1']]
    out0, out1 = pl.pallas_call(
        _mace_body,
        out_shape=[
            jax.ShapeDtypeStruct((_N, _F0), jnp.float32),
            jax.ShapeDtypeStruct((3, _N, _F1), jnp.float32),
        ],
    )(*args)
    return out0, jnp.transpose(out1, (1, 2, 0))


# speedup vs baseline: 755.0057x; 1.4363x over previous
import jax
import jax.numpy as jnp
from jax.experimental import pallas as pl

_N = 512
_F0 = 64
_F1 = 16
_NB = 8
_RMAX = 5.0
_AVG = 511.0
_PI = 3.141592653589793


def _dot(a, b):
    return jax.lax.dot_general(
        a, b, (((1,), (0,)), ((), ())),
        preferred_element_type=jnp.float32,
        precision=jax.lax.Precision.HIGHEST)


def _dot16(a, b):
    return jax.lax.dot_general(
        a.astype(jnp.bfloat16), b.astype(jnp.bfloat16),
        (((1,), (0,)), ((), ())),
        preferred_element_type=jnp.float32)


def _mace_body(x_ref, xt_ref, *refs):
    (wemb,
     wr0_0, wr1_0, wr2_0, wsc_0, ws0_0, wu0_0, wn_0, wg_0, wh1_0, wu1_0,
     wr0_1, wr1_1, wr2_1, wsc_1, ws0_1, wu0_1, wn_1, wg_1, wh1_1, wu1_1,
     wro0, wro1, out0_ref, out1_ref) = refs
    f32 = jnp.float32

    vec = []
    for c in range(3):
        col = x_ref[:, c:c + 1]
        row = xt_ref[c:c + 1, :]
        vec.append(col - row)
    d2 = vec[0] * vec[0] + vec[1] * vec[1] + vec[2] * vec[2] + 1e-12
    d = jnp.sqrt(d2)
    dinv = 1.0 / (d + 1e-9)
    u = jnp.clip(d / _RMAX, 0.0, 1.0 - 1e-6)
    env = jnp.where(d < _RMAX, jnp.exp(1.0 - 1.0 / (1.0 - u * u)), 0.0)
    ii = jax.lax.broadcasted_iota(jnp.int32, (_N, _N), 0)
    jj = jax.lax.broadcasted_iota(jnp.int32, (_N, _N), 1)
    pref = jnp.where(ii == jj, 0.0, jnp.sqrt(2.0 / _RMAX) * env * dinv)
    rb = [pref * jnp.sin(((b + 1) * _PI / _RMAX) * d) for b in range(_NB)]
    yhat = [v * dinv for v in vec]

    h0e = wemb[:]
    sc0 = _dot(h0e, wsc_0[:])
    wr0v, wr1v = wr0_0[:], wr1_0[:]
    S0 = jnp.zeros((_N, _F0), f32)
    t1 = [jnp.zeros((_N, _F1), f32) for _ in range(3)]
    for b in range(_NB):
        rbs = jnp.sum(rb[b], axis=1, keepdims=True)
        S0 = S0 + rbs * wr0v[b:b + 1, :]
        for c in range(3):
            rys = jnp.sum(rb[b] * yhat[c], axis=1, keepdims=True)
            t1[c] = t1[c] + rys * wr1v[b:b + 1, :]
    H0 = S0 * h0e * (1.0 / _AVG)
    H1 = [t1[c] * sc0 * (1.0 / _AVG) for c in range(3)]
    norms = H1[0] * H1[0] + H1[1] * H1[1] + H1[2] * H1[2]
    pre = (_dot(h0e, ws0_0[:]) + _dot(H0, wu0_0[:]) + _dot(norms, wn_0[:]))
    h0 = pre * jax.nn.sigmoid(pre)
    gate = jax.nn.sigmoid(_dot(h0, wg_0[:]))
    h1 = [_dot(H1[c], wu1_0[:]) * gate for c in range(3)]

    sc = _dot(h0, wsc_1[:])
    rhs = jnp.concatenate([h0, h1[0], h1[1], h1[2]], axis=1)
    wr0v, wr1v, wr2v = wr0_1[:], wr1_1[:], wr2_1[:]
    H0 = jnp.zeros((_N, _F0), f32)
    t1 = [jnp.zeros((_N, _F1), f32) for _ in range(3)]
    t2 = [jnp.zeros((_N, _F1), f32) for _ in range(3)]
    rhs16 = rhs.astype(jnp.bfloat16)
    sc16 = sc.astype(jnp.bfloat16)
    for b in range(_NB):
        Tb = _dot16(rb[b], rhs16)
        H0 = H0 + wr0v[b:b + 1, :] * Tb[:, :_F0]
        for c in range(3):
            lo = _F0 + _F1 * c
            t2[c] = t2[c] + wr2v[b:b + 1, :] * Tb[:, lo:lo + _F1]
            Mbc = _dot16(rb[b] * yhat[c], sc16)
            t1[c] = t1[c] + wr1v[b:b + 1, :] * Mbc
    H0 = H0 * (1.0 / _AVG)
    H1 = [(t1[c] + t2[c]) * (1.0 / _AVG) for c in range(3)]
    norms = H1[0] * H1[0] + H1[1] * H1[1] + H1[2] * H1[2]
    pre = (_dot(h0, ws0_1[:]) + _dot(H0, wu0_1[:]) + _dot(norms, wn_1[:]))
    h0 = pre * jax.nn.sigmoid(pre)
    gate = jax.nn.sigmoid(_dot(h0, wg_1[:]))
    h1 = [(_dot(h1[c], wh1_1[:]) + _dot(H1[c], wu1_1[:])) * gate
          for c in range(3)]

    out0_ref[:, :] = _dot(h0, wro0[:])
    for c in range(3):
        com_c = jnp.mean(x_ref[:, c])
        out1_ref[c] = _dot(h1[c], wro1[:]) + com_c


def kernel(x, params):
    xt = x.T
    names = ['Wr0', 'Wr1', 'Wr2', 'Wsc', 'Ws0', 'Wu0', 'Wn', 'Wg', 'Wh1',
             'Wu1']
    args = [x, xt, params['Wemb']]
    for i in range(2):
        args += [params[nm + '_' + str(i)] for nm in names]
    args += [params['Wro0'], params['Wro1']]
    out0, out1 = pl.pallas_call(
        _mace_body,
        out_shape=[
            jax.ShapeDtypeStruct((_N, _F0), jnp.float32),
            jax.ShapeDtypeStruct((3, _N, _F1), jnp.float32),
        ],
    )(*args)
    return out0, jnp.transpose(out1, (1, 2, 0))


# speedup vs baseline: 1327.8740x; 1.7588x over previous
import jax
import jax.numpy as jnp
from jax.experimental import pallas as pl

_N = 512
_F0 = 64
_F1 = 16
_NB = 8
_RMAX = 5.0
_AVG = 511.0
_PI = 3.141592653589793


def _dot(a, b):
    return jax.lax.dot_general(
        a, b, (((1,), (0,)), ((), ())),
        preferred_element_type=jnp.float32,
        precision=jax.lax.Precision.HIGHEST)


def _dot16(a, b):
    return jax.lax.dot_general(
        a.astype(jnp.bfloat16), b.astype(jnp.bfloat16),
        (((1,), (0,)), ((), ())),
        preferred_element_type=jnp.float32)


def _mace_body(x_ref, xt_ref, *refs):
    (wemb,
     wr0_0, wr1_0, wr2_0, wsc_0, ws0_0, wu0_0, wn_0, wg_0, wh1_0, wu1_0,
     wr0_1, wr1_1, wr2_1, wsc_1, ws0_1, wu0_1, wn_1, wg_1, wh1_1, wu1_1,
     wro0, wro1, out0_ref, out1_ref) = refs
    f32 = jnp.float32

    vec = []
    for c in range(3):
        col = x_ref[:, c:c + 1]
        row = xt_ref[c:c + 1, :]
        vec.append(col - row)
    d2 = vec[0] * vec[0] + vec[1] * vec[1] + vec[2] * vec[2] + 1e-12
    d = jnp.sqrt(d2)
    dinv = 1.0 / (d + 1e-9)
    u = jnp.clip(d / _RMAX, 0.0, 1.0 - 1e-6)
    env = jnp.where(d < _RMAX, jnp.exp(1.0 - 1.0 / (1.0 - u * u)), 0.0)
    ii = jax.lax.broadcasted_iota(jnp.int32, (_N, _N), 0)
    jj = jax.lax.broadcasted_iota(jnp.int32, (_N, _N), 1)
    pref = jnp.where(ii == jj, 0.0, jnp.sqrt(2.0 / _RMAX) * env * dinv)
    theta = (_PI / _RMAX) * d
    s1 = jnp.sin(theta)
    c2 = 2.0 * jnp.cos(theta)
    sins = [s1, c2 * s1]
    for _ in range(2, _NB):
        sins.append(c2 * sins[-1] - sins[-2])
    rb = [pref * sins[b] for b in range(_NB)]
    yhat = [v * dinv for v in vec]

    h0e = wemb[:]
    sc0 = _dot(h0e, wsc_0[:])
    wr0v, wr1v = wr0_0[:], wr1_0[:]
    S0 = jnp.zeros((_N, _F0), f32)
    t1 = [jnp.zeros((_N, _F1), f32) for _ in range(3)]
    for b in range(_NB):
        rbs = jnp.sum(rb[b], axis=1, keepdims=True)
        S0 = S0 + rbs * wr0v[b:b + 1, :]
        for c in range(3):
            rys = jnp.sum(rb[b] * yhat[c], axis=1, keepdims=True)
            t1[c] = t1[c] + rys * wr1v[b:b + 1, :]
    H0 = S0 * h0e * (1.0 / _AVG)
    H1 = [t1[c] * sc0 * (1.0 / _AVG) for c in range(3)]
    norms = H1[0] * H1[0] + H1[1] * H1[1] + H1[2] * H1[2]
    pre = (_dot(h0e, ws0_0[:]) + _dot(H0, wu0_0[:]) + _dot(norms, wn_0[:]))
    h0 = pre * jax.nn.sigmoid(pre)
    gate = jax.nn.sigmoid(_dot(h0, wg_0[:]))
    h1 = [_dot(H1[c], wu1_0[:]) * gate for c in range(3)]

    sc = _dot(h0, wsc_1[:])
    rhs = jnp.concatenate([h0, h1[0], h1[1], h1[2]], axis=1)
    wr0v, wr1v, wr2v = wr0_1[:], wr1_1[:], wr2_1[:]
    H0 = jnp.zeros((_N, _F0), f32)
    t1 = [jnp.zeros((_N, _F1), f32) for _ in range(3)]
    t2 = [jnp.zeros((_N, _F1), f32) for _ in range(3)]
    rhs16 = rhs.astype(jnp.bfloat16)
    sc16 = sc.astype(jnp.bfloat16)
    for b in range(_NB):
        Tb = _dot16(rb[b], rhs16)
        H0 = H0 + wr0v[b:b + 1, :] * Tb[:, :_F0]
        for c in range(3):
            lo = _F0 + _F1 * c
            t2[c] = t2[c] + wr2v[b:b + 1, :] * Tb[:, lo:lo + _F1]
            Mbc = _dot16(rb[b] * yhat[c], sc16)
            t1[c] = t1[c] + wr1v[b:b + 1, :] * Mbc
    H0 = H0 * (1.0 / _AVG)
    H1 = [(t1[c] + t2[c]) * (1.0 / _AVG) for c in range(3)]
    norms = H1[0] * H1[0] + H1[1] * H1[1] + H1[2] * H1[2]
    pre = (_dot(h0, ws0_1[:]) + _dot(H0, wu0_1[:]) + _dot(norms, wn_1[:]))
    h0 = pre * jax.nn.sigmoid(pre)
    gate = jax.nn.sigmoid(_dot(h0, wg_1[:]))
    h1 = [(_dot(h1[c], wh1_1[:]) + _dot(H1[c], wu1_1[:])) * gate
          for c in range(3)]

    out0_ref[:, :] = _dot(h0, wro0[:])
    for c in range(3):
        com_c = jnp.mean(x_ref[:, c])
        out1_ref[c] = _dot(h1[c], wro1[:]) + com_c


def kernel(x, params):
    xt = x.T
    names = ['Wr0', 'Wr1', 'Wr2', 'Wsc', 'Ws0', 'Wu0', 'Wn', 'Wg', 'Wh1',
             'Wu1']
    args = [x, xt, params['Wemb']]
    for i in range(2):
        args += [params[nm + '_' + str(i)] for nm in names]
    args += [params['Wro0'], params['Wro1']]
    out0, out1 = pl.pallas_call(
        _mace_body,
        out_shape=[
            jax.ShapeDtypeStruct((_N, _F0), jnp.float32),
            jax.ShapeDtypeStruct((3, _N, _F1), jnp.float32),
        ],
    )(*args)
    return out0, jnp.transpose(out1, (1, 2, 0))
